# baseline (speedup 1.0000x reference)
import functools

import jax
import jax.numpy as jnp
from jax import lax
from jax.experimental import pallas as pl
from jax.experimental.pallas import tpu as pltpu
from jax.experimental.pallas import tpu_sc as plsc

N = 10000
E = 320000
D = 128
DW = 144
L = 16
DL = 8
NC = 2
NS = 16
NW = NC * NS
EPW = E // NW
K = 16
NB = EPW // K
NBC = 125
NCHUNK = NB // NBC
NBUF = 5
GRP = 5
NG = NB // GRP
GPC = NBC // GRP
RPT = 624


def _mm_body(x_ref, w_ref, aw_ref, h_ref, af_ref):
    h = jnp.dot(x_ref[...], w_ref[...], preferred_element_type=jnp.float32)
    h_ref[...] = h
    af_ref[...] = jnp.dot(h, aw_ref[...], preferred_element_type=jnp.float32)


def _matmul(x, W, attW):
    Bn = 2000
    return pl.pallas_call(
        _mm_body,
        grid=(N // Bn,),
        in_specs=[
            pl.BlockSpec((Bn, D), lambda i: (i, 0)),
            pl.BlockSpec((D, D), lambda i: (0, 0)),
            pl.BlockSpec((D, DL), lambda i: (0, 0)),
        ],
        out_specs=[
            pl.BlockSpec((Bn, D), lambda i: (i, 0)),
            pl.BlockSpec((Bn, DL), lambda i: (i, 0)),
        ],
        out_shape=[
            jax.ShapeDtypeStruct((N, D), jnp.float32),
            jax.ShapeDtypeStruct((N, DL), jnp.float32),
        ],
    )(x, W, attW)


def _w_body(src_hbm, dst_hbm, asrc_hbm, adst_hbm, w_out,
            src_v, dst_v, asrc_v, adst_v, w_v):
    c = lax.axis_index("c")
    s = lax.axis_index("s")
    wid = s * NC + c
    pltpu.sync_copy(src_hbm.at[wid], src_v)
    pltpu.sync_copy(dst_hbm.at[wid], dst_v)
    pltpu.sync_copy(asrc_hbm, asrc_v)
    pltpu.sync_copy(adst_hbm, adst_v)

    def _b(i, _):
        si = src_v[i, :]
        di = dst_v[i, :]
        e = plsc.load_gather(asrc_v, [si]) + plsc.load_gather(adst_v, [di])
        e = jnp.where(e > 0.0, e, 0.2 * e)
        w_v[i, :] = jnp.exp(e)
        return 0
    lax.fori_loop(0, NB, _b, 0)
    pltpu.sync_copy(w_v, w_out.at[wid])


def _w_call(src3, dst3, a_src, a_dst):
    mesh = plsc.VectorSubcoreMesh(core_axis_name="c", subcore_axis_name="s")
    f = pl.kernel(
        _w_body,
        out_type=jax.ShapeDtypeStruct((NW, NB, K), jnp.float32),
        mesh=mesh,
        scratch_types=[
            pltpu.VMEM((NB, K), jnp.int32),
            pltpu.VMEM((NB, K), jnp.int32),
            pltpu.VMEM((N,), jnp.float32),
            pltpu.VMEM((N,), jnp.float32),
            pltpu.VMEM((NB, K), jnp.float32),
        ],
        compiler_params=pltpu.CompilerParams(needs_layout_passes=False,
                                             use_tc_tiling_on_sc=False),
    )
    return f(src3, dst3, a_src, a_dst)


def _lane_bcast(v, k):
    dnums = lax.GatherDimensionNumbers(
        offset_dims=(), collapsed_slice_dims=(0,), start_index_map=(0,))
    return lax.gather(v, jnp.full((L, 1), k, jnp.int32), dnums, (1,),
                      mode=lax.GatherScatterMode.PROMISE_IN_BOUNDS)


def _sc_body(h_hbm, src_hbm, dst_hbm, w_hbm,
             acc_out,
             src_r, dst_r, w_r, grows, srows,
             acc_sp, gsem, ssem, csem):
    c = lax.axis_index("c")
    s = lax.axis_index("s")
    wid = s * NC + c

    zf = jnp.zeros((L,), jnp.float32)
    lane_ids = jnp.arange(L, dtype=jnp.int32)
    for i in range(16):
        for j in range(DW // L):
            srows[0, i, pl.ds(j * L, L)] = zf
    for q in range(1, NBUF):
        for col in range(D, DW):
            plsc.store_scatter(srows.at[q],
                               [lane_ids, jnp.full((L,), col, jnp.int32)], zf)

    nz = jnp.where(s == NS - 1, RPT // 16 + 1, RPT // 16)

    def _zero(i, _):
        off = s * RPT + i * 16
        pltpu.sync_copy(srows.at[0], acc_sp.at[pl.ds(off, 16)])
        return 0
    lax.fori_loop(0, nz, _zero, 0)

    plsc.subcore_barrier()

    wcol_ids = jnp.full((L,), D, jnp.int32)

    pltpu.sync_copy(src_hbm.at[wid, pl.ds(0, NBC)], src_r.at[0])
    pltpu.sync_copy(dst_hbm.at[wid, pl.ds(0, NBC)], dst_r.at[0])
    pltpu.sync_copy(w_hbm.at[wid, pl.ds(0, NBC)], w_r.at[0])
    for t in range(NBUF):
        si0 = src_r[0, t, :]
        pltpu.async_copy(h_hbm.at[si0], grows.at[t], gsem.at[t])

    def _group(g, _):
        gm = g % GPC
        ck = g // GPC
        p = ck % 2

        @pl.when(jnp.logical_and(gm == 0, ck < NCHUNK - 1))
        def _issue_chunk():
            off = (ck + 1) * NBC
            pltpu.async_copy(src_hbm.at[wid, pl.ds(off, NBC)],
                             src_r.at[1 - p], csem)
            pltpu.async_copy(dst_hbm.at[wid, pl.ds(off, NBC)],
                             dst_r.at[1 - p], csem)
            pltpu.async_copy(w_hbm.at[wid, pl.ds(off, NBC)],
                             w_r.at[1 - p], csem)

        @pl.when(jnp.logical_and(gm == GPC - 1, ck < NCHUNK - 1))
        def _wait_chunk():
            off = (ck + 1) * NBC
            pltpu.make_async_copy(src_hbm.at[wid, pl.ds(off, NBC)],
                                  src_r.at[1 - p], csem).wait()
            pltpu.make_async_copy(dst_hbm.at[wid, pl.ds(off, NBC)],
                                  dst_r.at[1 - p], csem).wait()
            pltpu.make_async_copy(w_hbm.at[wid, pl.ds(off, NBC)],
                                  w_r.at[1 - p], csem).wait()

        for q in range(GRP):
            t = g * GRP + q
            rt = gm * GRP + q
            si = src_r[p, rt, :]
            di = dst_r[p, rt, :]
            w = w_r[p, rt, :]
            pltpu.make_async_copy(h_hbm.at[si], grows.at[q],
                                  gsem.at[q]).wait()
            @pl.when(g > 0)
            def _wait_scatter():
                pltpu.make_async_copy(srows.at[q], acc_sp.at[di],
                                      ssem.at[q]).wait()
            plsc.store_scatter(srows.at[q], [lane_ids, wcol_ids], w)
            wbs = [_lane_bcast(w, k) for k in range(K)]
            for k in range(K):
                for j in range(D // L):
                    srows[q, k, pl.ds(j * L, L)] = \
                        grows[q, k, pl.ds(j * L, L)] * wbs[k]
            pltpu.async_copy(srows.at[q], acc_sp.at[di], ssem.at[q],
                             add=True)

            @pl.when(g < NG - 1)
            def _issue_gather():
                t5 = t + NBUF
                c5 = t5 // NBC
                p5 = c5 % 2
                r5 = t5 - c5 * NBC
                si5 = src_r[p5, r5, :]
                pltpu.async_copy(h_hbm.at[si5], grows.at[q], gsem.at[q])
        return 0

    lax.fori_loop(0, NG, _group, 0)

    for q in range(NBUF):
        di = dst_r[(NB - 1) // NBC % 2, NBC - NBUF + q, :]
        pltpu.make_async_copy(srows.at[q], acc_sp.at[di], ssem.at[q]).wait()

    plsc.subcore_barrier()

    pltpu.sync_copy(acc_sp.at[pl.ds(s * RPT, RPT)],
                    acc_out.at[c, pl.ds(s * RPT, RPT)])

    @pl.when(s == NS - 1)
    def _tail():
        pltpu.sync_copy(acc_sp.at[pl.ds(NS * RPT, N - NS * RPT)],
                        acc_out.at[c, pl.ds(NS * RPT, N - NS * RPT)])


def _sc_call(h, src3, dst3, w3):
    mesh = plsc.VectorSubcoreMesh(core_axis_name="c", subcore_axis_name="s")
    f = pl.kernel(
        _sc_body,
        out_type=jax.ShapeDtypeStruct((NC, N, DW), jnp.float32),
        mesh=mesh,
        scratch_types=[
            pltpu.VMEM((2, NBC, K), jnp.int32),
            pltpu.VMEM((2, NBC, K), jnp.int32),
            pltpu.VMEM((2, NBC, K), jnp.float32),
            pltpu.VMEM((NBUF, K, D), jnp.float32),
            pltpu.VMEM((NBUF, K, DW), jnp.float32),
            pltpu.VMEM_SHARED((N, DW), jnp.float32),
            pltpu.SemaphoreType.DMA((NBUF,)),
            pltpu.SemaphoreType.DMA((NBUF,)),
            pltpu.SemaphoreType.DMA,
        ],
        compiler_params=pltpu.CompilerParams(needs_layout_passes=False,
                                             use_tc_tiling_on_sc=False),
    )
    return f(h, src3, dst3, w3)


def _fin_body(acc_ref, b_ref, o_ref):
    a = acc_ref[0, :, 0:D] + acc_ref[1, :, 0:D]
    d = acc_ref[0, :, D:D + 1] + acc_ref[1, :, D:D + 1]
    d = jnp.where(d > 0.0, d, 1.0)
    o_ref[...] = jnp.maximum(a / d + b_ref[...], 0.0)


def _finalize(acc, bias2d):
    Bn = 1000
    return pl.pallas_call(
        _fin_body,
        grid=(N // Bn,),
        in_specs=[
            pl.BlockSpec((NC, Bn, DW), lambda i: (0, i, 0)),
            pl.BlockSpec((1, D), lambda i: (0, 0)),
        ],
        out_specs=pl.BlockSpec((Bn, D), lambda i: (i, 0)),
        out_shape=jax.ShapeDtypeStruct((N, D), jnp.float32),
    )(acc, bias2d)


def kernel(x, edge_index, W, att_src, att_dst, bias):
    attW = (jnp.zeros((D, DL), jnp.float32)
            .at[:, 0].set(att_src)
            .at[:, 1].set(att_dst))
    h, af = _matmul(x, W, attW)
    a_src = af[:, 0]
    a_dst = af[:, 1]
    src3 = edge_index[0].reshape(NW, NB, K)
    dst3 = edge_index[1].reshape(NW, NB, K)
    w3 = _w_call(src3, dst3, a_src, a_dst)
    acc = _sc_call(h, src3, dst3, w3)
    return _finalize(acc, bias.reshape(1, D))


# speedup vs baseline: 1.1138x; 1.1138x over previous
import functools

import jax
import jax.numpy as jnp
from jax import lax
from jax.experimental import pallas as pl
from jax.experimental.pallas import tpu as pltpu
from jax.experimental.pallas import tpu_sc as plsc

N = 10000
E = 320000
D = 128
L = 16
DL = 8
NC = 2
NS = 16
NW = NC * NS
EPW = E // NW
K = 16
NB = EPW // K
NBC = 25
NCHUNK = NB // NBC
NBUF = 5
NG = NB // NBUF
RPT = 624


def _mm_body(x_ref, w_ref, aw_ref, h_ref, af_ref):
    h = jnp.dot(x_ref[...], w_ref[...], preferred_element_type=jnp.float32)
    h_ref[...] = h
    af_ref[...] = jnp.dot(h, aw_ref[...], preferred_element_type=jnp.float32)


def _matmul(x, W, attW):
    Bn = 1000
    return pl.pallas_call(
        _mm_body,
        grid=(N // Bn,),
        in_specs=[
            pl.BlockSpec((Bn, D), lambda i: (i, 0)),
            pl.BlockSpec((D, D), lambda i: (0, 0)),
            pl.BlockSpec((D, D), lambda i: (0, 0)),
        ],
        out_specs=[
            pl.BlockSpec((Bn, D), lambda i: (i, 0)),
            pl.BlockSpec((Bn, D), lambda i: (i, 0)),
        ],
        out_shape=[
            jax.ShapeDtypeStruct((N, D), jnp.float32),
            jax.ShapeDtypeStruct((N, D), jnp.float32),
        ],
    )(x, W, attW)


def _lane_bcast(v, k):
    dnums = lax.GatherDimensionNumbers(
        offset_dims=(), collapsed_slice_dims=(0,), start_index_map=(0,))
    return lax.gather(v, jnp.full((L, 1), k, jnp.int32), dnums, (1,),
                      mode=lax.GatherScatterMode.PROMISE_IN_BOUNDS)


def _sc_body(h_hbm, src_hbm, dst_hbm, asrc_hbm, adst_hbm,
             acc_out, den_out,
             src_r, dst_r, asrc_v, adst_v, grows, srows, wtile,
             acc_sp, den_sp, gsem, ssem, dsem, csem):
    c = lax.axis_index("c")
    s = lax.axis_index("s")
    wid = s * NC + c

    pltpu.sync_copy(asrc_hbm, asrc_v)
    pltpu.sync_copy(adst_hbm, adst_v)

    zf = jnp.zeros((L,), jnp.float32)
    lane_ids = jnp.arange(L, dtype=jnp.int32)
    for i in range(16):
        for j in range(D // L):
            grows[0, i, pl.ds(j * L, L)] = zf
    for q in range(NBUF):
        for col in range(DL):
            plsc.store_scatter(wtile.at[q],
                               [lane_ids, jnp.full((L,), col, jnp.int32)], zf)

    nz = jnp.where(s == NS - 1, RPT // 16 + 1, RPT // 16)

    def _zero(i, _):
        off = s * RPT + i * 16
        pltpu.sync_copy(grows.at[0], acc_sp.at[pl.ds(off, 16)])
        pltpu.sync_copy(wtile.at[0], den_sp.at[pl.ds(off, 16)])
        return 0
    lax.fori_loop(0, nz, _zero, 0)

    plsc.subcore_barrier()

    zero_ids = jnp.zeros((L,), jnp.int32)

    pltpu.sync_copy(src_hbm.at[wid, pl.ds(0, NBC)], src_r.at[0])
    pltpu.sync_copy(dst_hbm.at[wid, pl.ds(0, NBC)], dst_r.at[0])
    for t in range(NBUF):
        si0 = src_r[0, t, :]
        pltpu.async_copy(h_hbm.at[si0], grows.at[t], gsem.at[t])

    def _group(g, _):
        gm = g % 5
        ck = g // 5
        p = ck % 2

        @pl.when(jnp.logical_and(gm == 0, ck < NCHUNK - 1))
        def _issue_chunk():
            off = (ck + 1) * NBC
            pltpu.async_copy(src_hbm.at[wid, pl.ds(off, NBC)],
                             src_r.at[1 - p], csem)
            pltpu.async_copy(dst_hbm.at[wid, pl.ds(off, NBC)],
                             dst_r.at[1 - p], csem)

        @pl.when(jnp.logical_and(gm == 4, ck < NCHUNK - 1))
        def _wait_chunk():
            off = (ck + 1) * NBC
            pltpu.make_async_copy(src_hbm.at[wid, pl.ds(off, NBC)],
                                  src_r.at[1 - p], csem).wait()
            pltpu.make_async_copy(dst_hbm.at[wid, pl.ds(off, NBC)],
                                  dst_r.at[1 - p], csem).wait()

        for q in range(NBUF):
            t = g * NBUF + q
            rt = gm * NBUF + q
            si = src_r[p, rt, :]
            di = dst_r[p, rt, :]
            pltpu.make_async_copy(h_hbm.at[si], grows.at[q], gsem.at[q]).wait()
            @pl.when(g > 0)
            def _wait_scatter():
                pltpu.make_async_copy(srows.at[q], acc_sp.at[di],
                                      ssem.at[q]).wait()
                pltpu.make_async_copy(wtile.at[q], den_sp.at[di],
                                      dsem.at[q]).wait()
            a_s = plsc.load_gather(asrc_v, [si])
            a_d = plsc.load_gather(adst_v, [di])
            e = a_s + a_d
            e = jnp.where(e > 0.0, e, 0.2 * e)
            w = jnp.exp(e)
            plsc.store_scatter(wtile.at[q], [lane_ids, zero_ids], w)
            wbs = [_lane_bcast(w, k) for k in range(K)]
            for k in range(K):
                for j in range(D // L):
                    srows[q, k, pl.ds(j * L, L)] = \
                        grows[q, k, pl.ds(j * L, L)] * wbs[k]
            pltpu.async_copy(srows.at[q], acc_sp.at[di], ssem.at[q], add=True)
            pltpu.async_copy(wtile.at[q], den_sp.at[di], dsem.at[q], add=True)

            @pl.when(g < NG - 1)
            def _issue_gather():
                t5 = t + NBUF
                c5 = t5 // (NBC * 1)
                c5 = t5 // NBC
                p5 = c5 % 2
                r5 = t5 - c5 * NBC
                si5 = src_r[p5, r5, :]
                pltpu.async_copy(h_hbm.at[si5], grows.at[q], gsem.at[q])
        return 0

    lax.fori_loop(0, NG, _group, 0)

    for q in range(NBUF):
        di = dst_r[(NB - 1) // NBC % 2, NBC - NBUF + q, :]
        pltpu.make_async_copy(srows.at[q], acc_sp.at[di], ssem.at[q]).wait()
        pltpu.make_async_copy(wtile.at[q], den_sp.at[di], dsem.at[q]).wait()

    plsc.subcore_barrier()

    pltpu.sync_copy(acc_sp.at[pl.ds(s * RPT, RPT)],
                    acc_out.at[c, pl.ds(s * RPT, RPT)])
    pltpu.sync_copy(den_sp.at[pl.ds(s * RPT, RPT)],
                    den_out.at[c, pl.ds(s * RPT, RPT)])

    @pl.when(s == NS - 1)
    def _tail():
        pltpu.sync_copy(acc_sp.at[pl.ds(NS * RPT, N - NS * RPT)],
                        acc_out.at[c, pl.ds(NS * RPT, N - NS * RPT)])
        pltpu.sync_copy(den_sp.at[pl.ds(NS * RPT, N - NS * RPT)],
                        den_out.at[c, pl.ds(NS * RPT, N - NS * RPT)])


def _sc_call(h, src3, dst3, a_src, a_dst):
    mesh = plsc.VectorSubcoreMesh(core_axis_name="c", subcore_axis_name="s")
    f = pl.kernel(
        _sc_body,
        out_type=[
            jax.ShapeDtypeStruct((NC, N, D), jnp.float32),
            jax.ShapeDtypeStruct((NC, N, DL), jnp.float32),
        ],
        mesh=mesh,
        scratch_types=[
            pltpu.VMEM((2, NBC, K), jnp.int32),
            pltpu.VMEM((2, NBC, K), jnp.int32),
            pltpu.VMEM((N,), jnp.float32),
            pltpu.VMEM((N,), jnp.float32),
            pltpu.VMEM((NBUF, K, D), jnp.float32),
            pltpu.VMEM((NBUF, K, D), jnp.float32),
            pltpu.VMEM((NBUF, K, DL), jnp.float32),
            pltpu.VMEM_SHARED((N, D), jnp.float32),
            pltpu.VMEM_SHARED((N, DL), jnp.float32),
            pltpu.SemaphoreType.DMA((NBUF,)),
            pltpu.SemaphoreType.DMA((NBUF,)),
            pltpu.SemaphoreType.DMA((NBUF,)),
            pltpu.SemaphoreType.DMA,
        ],
        compiler_params=pltpu.CompilerParams(needs_layout_passes=False,
                                             use_tc_tiling_on_sc=False),
    )
    return f(h, src3, dst3, a_src, a_dst)


def _fin_body(acc_ref, den_ref, b_ref, o_ref):
    a = acc_ref[0] + acc_ref[1]
    d = den_ref[0, :, 0:1] + den_ref[1, :, 0:1]
    d = jnp.where(d > 0.0, d, 1.0)
    o_ref[...] = jnp.maximum(a / d + b_ref[...], 0.0)


def _finalize(acc, den, bias2d):
    Bn = 1000
    return pl.pallas_call(
        _fin_body,
        grid=(N // Bn,),
        in_specs=[
            pl.BlockSpec((NC, Bn, D), lambda i: (0, i, 0)),
            pl.BlockSpec((NC, Bn, DL), lambda i: (0, i, 0)),
            pl.BlockSpec((1, D), lambda i: (0, 0)),
        ],
        out_specs=pl.BlockSpec((Bn, D), lambda i: (i, 0)),
        out_shape=jax.ShapeDtypeStruct((N, D), jnp.float32),
    )(acc, den, bias2d)


def kernel(x, edge_index, W, att_src, att_dst, bias):
    attW = (jnp.zeros((D, D), jnp.float32)
            .at[:, 0].set(att_src)
            .at[:, 1].set(att_dst))
    h, af = _matmul(x, W, attW)
    a_src = af[:, 0]
    a_dst = af[:, 1]
    src3 = edge_index[0].reshape(NW, NB, K)
    dst3 = edge_index[1].reshape(NW, NB, K)
    acc, den = _sc_call(h, src3, dst3, a_src, a_dst)
    return _finalize(acc, den, bias.reshape(1, D))
